# Initial kernel scaffold; baseline (speedup 1.0000x reference)
import jax
import jax.numpy as jnp
from jax.experimental import pallas as pl


def kernel(x, edge_index, W_self0, W_neigh0, b0, W_self1, W_neigh1, b1, W_self2, W_neigh2, b2):
    raise NotImplementedError("write your pallas kernel here")


# speedup vs baseline: 3.9893x; 3.9893x over previous
import functools

import jax
import jax.numpy as jnp
from jax import lax
from jax.experimental import pallas as pl
from jax.experimental.pallas import tpu as pltpu
from jax.experimental.pallas import tpu_sc as plsc

N = 10000
NP = 10112
E = 320000
D = 128
K = 80
NC = 2
NS = 16
BR = 1000


def _sc_kernel(split_roles):
    mesh = plsc.VectorSubcoreMesh(core_axis_name="c", subcore_axis_name="s")

    @functools.partial(
        pl.kernel,
        mesh=mesh,
        out_type=jax.ShapeDtypeStruct((NC, NP, D), jnp.float32),
        scratch_types=[
            pltpu.VMEM((K,), jnp.int32),
            pltpu.VMEM((K,), jnp.int32),
            pltpu.VMEM((K, D), jnp.float32),
            pltpu.VMEM_SHARED((NP, D), jnp.float32),
            pltpu.SemaphoreType.DMA,
        ],
    )
    def k(src_hbm, dst_hbm, table_hbm, zeros_hbm, ones_hbm, out_hbm,
          src_v, dst_v, rows_v, acc_sh, sem):
        c = lax.axis_index("c")
        s = lax.axis_index("s")
        rows_per = NP // NS
        r0 = s * rows_per
        pltpu.sync_copy(zeros_hbm.at[pl.ds(r0, rows_per)],
                        acc_sh.at[pl.ds(r0, rows_per)])
        if split_roles:
            pltpu.sync_copy(ones_hbm, rows_v)
            per_tile = E // NS
            base0 = s * per_tile
        else:
            per_tile = E // (NC * NS)
            base0 = c * (E // NC) + s * per_tile
        plsc.subcore_barrier()

        def body(i, carry):
            base = base0 + i * K
            pltpu.sync_copy(dst_hbm.at[pl.ds(base, K)], dst_v)
            if split_roles:
                @pl.when(c == 0)
                def _():
                    pltpu.sync_copy(src_hbm.at[pl.ds(base, K)], src_v)
                    pltpu.async_copy(table_hbm.at[src_v], rows_v, sem).wait()
            else:
                pltpu.sync_copy(src_hbm.at[pl.ds(base, K)], src_v)
                pltpu.async_copy(table_hbm.at[src_v], rows_v, sem).wait()
            pltpu.sync_copy(rows_v, acc_sh.at[dst_v], add=True)
            return carry

        lax.fori_loop(0, per_tile // K, body, 0)
        plsc.subcore_barrier()
        pltpu.sync_copy(acc_sh.at[pl.ds(r0, rows_per)],
                        out_hbm.at[c, pl.ds(r0, rows_per)])

    return k


def _dot(a, b):
    return jnp.dot(a, b, preferred_element_type=jnp.float32,
                   precision=lax.Precision.HIGHEST)


def _table0_body(x_ref, wnt_ref, out_ref):
    out_ref[...] = _dot(x_ref[...], wnt_ref[...])


def _table0(x, wnt):
    return pl.pallas_call(
        _table0_body,
        grid=(N // BR,),
        in_specs=[
            pl.BlockSpec((BR, D), lambda i: (i, 0)),
            pl.BlockSpec((D, D), lambda i: (0, 0)),
        ],
        out_specs=pl.BlockSpec((BR, D), lambda i: (i, 0)),
        out_shape=jax.ShapeDtypeStruct((N, D), jnp.float32),
    )(x, wnt)


def _combine1_body(h_ref, p_ref, wst_ref, b_ref, wnt_ref,
                   h_out_ref, t_out_ref, dinv_out_ref):
    deginv = 1.0 / jnp.maximum(p_ref[1], 1.0)
    dinv_out_ref[...] = deginv
    z = _dot(h_ref[...], wst_ref[...]) + p_ref[0] * deginv + b_ref[...]
    h_next = jnp.maximum(z, 0.0)
    h_out_ref[...] = h_next
    t_out_ref[...] = _dot(h_next, wnt_ref[...])


def _combine1(h, partials, wst, b, wnt_next):
    return pl.pallas_call(
        _combine1_body,
        grid=(N // BR,),
        in_specs=[
            pl.BlockSpec((BR, D), lambda i: (i, 0)),
            pl.BlockSpec((NC, BR, D), lambda i: (0, i, 0)),
            pl.BlockSpec((D, D), lambda i: (0, 0)),
            pl.BlockSpec((1, D), lambda i: (0, 0)),
            pl.BlockSpec((D, D), lambda i: (0, 0)),
        ],
        out_specs=[
            pl.BlockSpec((BR, D), lambda i: (i, 0)),
            pl.BlockSpec((BR, D), lambda i: (i, 0)),
            pl.BlockSpec((BR, D), lambda i: (i, 0)),
        ],
        out_shape=[
            jax.ShapeDtypeStruct((N, D), jnp.float32),
            jax.ShapeDtypeStruct((N, D), jnp.float32),
            jax.ShapeDtypeStruct((N, D), jnp.float32),
        ],
    )(h, partials, wst, b, wnt_next)


def _combine2_body(h_ref, p_ref, dinv_ref, wst_ref, b_ref, wnt_ref,
                   h_out_ref, t_out_ref):
    p = p_ref[0] + p_ref[1]
    z = _dot(h_ref[...], wst_ref[...]) + p * dinv_ref[...] + b_ref[...]
    h_next = jnp.maximum(z, 0.0)
    h_out_ref[...] = h_next
    t_out_ref[...] = _dot(h_next, wnt_ref[...])


def _combine2(h, partials, dinv, wst, b, wnt_next):
    return pl.pallas_call(
        _combine2_body,
        grid=(N // BR,),
        in_specs=[
            pl.BlockSpec((BR, D), lambda i: (i, 0)),
            pl.BlockSpec((NC, BR, D), lambda i: (0, i, 0)),
            pl.BlockSpec((BR, D), lambda i: (i, 0)),
            pl.BlockSpec((D, D), lambda i: (0, 0)),
            pl.BlockSpec((1, D), lambda i: (0, 0)),
            pl.BlockSpec((D, D), lambda i: (0, 0)),
        ],
        out_specs=[
            pl.BlockSpec((BR, D), lambda i: (i, 0)),
            pl.BlockSpec((BR, D), lambda i: (i, 0)),
        ],
        out_shape=[
            jax.ShapeDtypeStruct((N, D), jnp.float32),
            jax.ShapeDtypeStruct((N, D), jnp.float32),
        ],
    )(h, partials, dinv, wst, b, wnt_next)


def _final_body(h_ref, p_ref, dinv_ref, wst_ref, b_ref, out_ref):
    p = p_ref[0] + p_ref[1]
    out_ref[...] = (_dot(h_ref[...], wst_ref[...]) + p * dinv_ref[...]
                    + b_ref[...])


def _final(h, partials, dinv, wst, b):
    return pl.pallas_call(
        _final_body,
        grid=(N // BR,),
        in_specs=[
            pl.BlockSpec((BR, D), lambda i: (i, 0)),
            pl.BlockSpec((NC, BR, D), lambda i: (0, i, 0)),
            pl.BlockSpec((BR, D), lambda i: (i, 0)),
            pl.BlockSpec((D, D), lambda i: (0, 0)),
            pl.BlockSpec((1, D), lambda i: (0, 0)),
        ],
        out_specs=pl.BlockSpec((BR, D), lambda i: (i, 0)),
        out_shape=jax.ShapeDtypeStruct((N, D), jnp.float32),
    )(h, partials, dinv, wst, b)


def kernel(x, edge_index, W_self0, W_neigh0, b0,
           W_self1, W_neigh1, b1, W_self2, W_neigh2, b2):
    src = edge_index[0]
    dst = edge_index[1]
    zeros = jnp.zeros((NP, D), jnp.float32)
    ones = jnp.ones((K, D), jnp.float32)

    sc_agg0 = _sc_kernel(split_roles=True)
    sc_agg = _sc_kernel(split_roles=False)

    t0 = _table0(x, W_neigh0.T)
    p1 = sc_agg0(src, dst, t0, zeros, ones)
    h1, t1, dinv = _combine1(x, p1, W_self0.T, b0.reshape(1, D), W_neigh1.T)
    p2 = sc_agg(src, dst, t1, zeros, ones)
    h2, t2 = _combine2(h1, p2, dinv, W_self1.T, b1.reshape(1, D), W_neigh2.T)
    p3 = sc_agg(src, dst, t2, zeros, ones)
    return _final(h2, p3, dinv, W_self2.T, b2.reshape(1, D))


# speedup vs baseline: 4.5765x; 1.1472x over previous
import functools

import jax
import jax.numpy as jnp
from jax import lax
from jax.experimental import pallas as pl
from jax.experimental.pallas import tpu as pltpu
from jax.experimental.pallas import tpu_sc as plsc

N = 10000
NP = 10112
E = 320000
D = 128
K = 40
NC = 2
NS = 16
NBUF = 5
BR = 1000

_PER_TILE = E // (NC * NS)
_NCHUNK = _PER_TILE // K
_NGROUP = _NCHUNK // NBUF
_RPS = NP // NS


def _sc_mesh():
    return plsc.VectorSubcoreMesh(core_axis_name="c", subcore_axis_name="s")


def _feature_phase(src_hbm, dst_hbm, table_hbm, srcv, dstv, rows, acc,
                   gsem, base0):
    def idx_load(i, b):
        base = base0 + i * K
        pltpu.sync_copy(src_hbm.at[pl.ds(base, K)], srcv[b])
        pltpu.sync_copy(dst_hbm.at[pl.ds(base, K)], dstv[b])

    def gather_start(b):
        pltpu.async_copy(table_hbm.at[srcv[b]], rows.at[b], gsem)

    def gather_wait(b):
        pltpu.make_async_copy(table_hbm.at[srcv[b]], rows.at[b], gsem).wait()

    for b in range(NBUF):
        idx_load(b, b)
        gather_start(b)

    def group(g, carry):
        for b in range(NBUF):
            gather_wait(b)
            pltpu.sync_copy(rows.at[b], acc.at[dstv[b]], add=True)
            idx_load(g * NBUF + b, b)
            gather_start(b)
        return carry

    lax.fori_loop(1, _NGROUP, group, 0)

    for b in range(NBUF):
        gather_wait(b)
        pltpu.sync_copy(rows.at[b], acc.at[dstv[b]], add=True)


def _deg_phase(dst_hbm, dstv, ones_v, acc, base0):
    def chunk(i, b):
        pltpu.sync_copy(dst_hbm.at[pl.ds(base0 + i * K, K)], dstv[b])
        pltpu.sync_copy(ones_v, acc.at[dstv[b]], add=True)

    def group(g, carry):
        for b in range(NBUF):
            chunk(g * NBUF + b, b)
        return carry

    lax.fori_loop(0, _NGROUP, group, 0)


_AGG_SCRATCH = (
    [pltpu.VMEM((K,), jnp.int32) for _ in range(2 * NBUF)]
    + [
        pltpu.VMEM((NBUF, K, D), jnp.float32),
        pltpu.VMEM_SHARED((NP, D), jnp.float32),
        pltpu.SemaphoreType.DMA,
    ]
)


def _agg_kernel():
    @functools.partial(
        pl.kernel,
        mesh=_sc_mesh(),
        out_type=jax.ShapeDtypeStruct((NC, NP, D), jnp.float32),
        scratch_types=list(_AGG_SCRATCH),
    )
    def k(src_hbm, dst_hbm, table_hbm, zeros_hbm, out_hbm,
          s0, s1, s2, s3, s4, d0, d1, d2, d3, d4,
          rows, acc, gsem):
        srcv = (s0, s1, s2, s3, s4)
        dstv = (d0, d1, d2, d3, d4)
        c = lax.axis_index("c")
        s = lax.axis_index("s")
        r0 = s * _RPS
        sl = pl.ds(r0, _RPS)
        pltpu.sync_copy(zeros_hbm.at[sl], acc.at[sl])
        plsc.subcore_barrier()
        base0 = c * (E // NC) + s * _PER_TILE
        _feature_phase(src_hbm, dst_hbm, table_hbm, srcv, dstv, rows, acc,
                       gsem, base0)
        plsc.subcore_barrier()
        pltpu.sync_copy(acc.at[sl], out_hbm.at[c, sl])

    return k


def _agg_deg_kernel():
    @functools.partial(
        pl.kernel,
        mesh=_sc_mesh(),
        out_type=[
            jax.ShapeDtypeStruct((NC, NP, D), jnp.float32),
            jax.ShapeDtypeStruct((NC, NP, D), jnp.float32),
        ],
        scratch_types=list(_AGG_SCRATCH) + [pltpu.VMEM((K, D), jnp.float32)],
    )
    def k(src_hbm, dst_hbm, table_hbm, zeros_hbm, ones_hbm,
          out_hbm, outdeg_hbm,
          s0, s1, s2, s3, s4, d0, d1, d2, d3, d4,
          rows, acc, gsem, ones_v):
        srcv = (s0, s1, s2, s3, s4)
        dstv = (d0, d1, d2, d3, d4)
        c = lax.axis_index("c")
        s = lax.axis_index("s")
        r0 = s * _RPS
        sl = pl.ds(r0, _RPS)
        pltpu.sync_copy(zeros_hbm.at[sl], acc.at[sl])
        pltpu.sync_copy(ones_hbm, ones_v)
        plsc.subcore_barrier()
        base0 = c * (E // NC) + s * _PER_TILE
        _feature_phase(src_hbm, dst_hbm, table_hbm, srcv, dstv, rows, acc,
                       gsem, base0)
        plsc.subcore_barrier()
        pltpu.sync_copy(acc.at[sl], out_hbm.at[c, sl])
        pltpu.sync_copy(zeros_hbm.at[sl], acc.at[sl])
        plsc.subcore_barrier()
        _deg_phase(dst_hbm, dstv, ones_v, acc, base0)
        plsc.subcore_barrier()
        pltpu.sync_copy(acc.at[sl], outdeg_hbm.at[c, sl])

    return k


def _dot(a, b):
    return jnp.dot(a, b, preferred_element_type=jnp.float32,
                   precision=lax.Precision.HIGHEST)


def _table0_body(x_ref, wnt_ref, out_ref):
    out_ref[...] = _dot(x_ref[...], wnt_ref[...])


def _table0(x, wnt):
    return pl.pallas_call(
        _table0_body,
        grid=(N // BR,),
        in_specs=[
            pl.BlockSpec((BR, D), lambda i: (i, 0)),
            pl.BlockSpec((D, D), lambda i: (0, 0)),
        ],
        out_specs=pl.BlockSpec((BR, D), lambda i: (i, 0)),
        out_shape=jax.ShapeDtypeStruct((N, D), jnp.float32),
    )(x, wnt)


def _combine1_body(h_ref, p_ref, pd_ref, wst_ref, b_ref, wnt_ref,
                   h_out_ref, t_out_ref, dinv_out_ref):
    deginv = 1.0 / jnp.maximum(pd_ref[0] + pd_ref[1], 1.0)
    dinv_out_ref[...] = deginv
    p = p_ref[0] + p_ref[1]
    z = _dot(h_ref[...], wst_ref[...]) + p * deginv + b_ref[...]
    h_next = jnp.maximum(z, 0.0)
    h_out_ref[...] = h_next
    t_out_ref[...] = _dot(h_next, wnt_ref[...])


def _combine1(h, partials, deg_partials, wst, b, wnt_next):
    return pl.pallas_call(
        _combine1_body,
        grid=(N // BR,),
        in_specs=[
            pl.BlockSpec((BR, D), lambda i: (i, 0)),
            pl.BlockSpec((NC, BR, D), lambda i: (0, i, 0)),
            pl.BlockSpec((NC, BR, D), lambda i: (0, i, 0)),
            pl.BlockSpec((D, D), lambda i: (0, 0)),
            pl.BlockSpec((1, D), lambda i: (0, 0)),
            pl.BlockSpec((D, D), lambda i: (0, 0)),
        ],
        out_specs=[
            pl.BlockSpec((BR, D), lambda i: (i, 0)),
            pl.BlockSpec((BR, D), lambda i: (i, 0)),
            pl.BlockSpec((BR, D), lambda i: (i, 0)),
        ],
        out_shape=[
            jax.ShapeDtypeStruct((N, D), jnp.float32),
            jax.ShapeDtypeStruct((N, D), jnp.float32),
            jax.ShapeDtypeStruct((N, D), jnp.float32),
        ],
    )(h, partials, deg_partials, wst, b, wnt_next)


def _combine2_body(h_ref, p_ref, dinv_ref, wst_ref, b_ref, wnt_ref,
                   h_out_ref, t_out_ref):
    p = p_ref[0] + p_ref[1]
    z = _dot(h_ref[...], wst_ref[...]) + p * dinv_ref[...] + b_ref[...]
    h_next = jnp.maximum(z, 0.0)
    h_out_ref[...] = h_next
    t_out_ref[...] = _dot(h_next, wnt_ref[...])


def _combine2(h, partials, dinv, wst, b, wnt_next):
    return pl.pallas_call(
        _combine2_body,
        grid=(N // BR,),
        in_specs=[
            pl.BlockSpec((BR, D), lambda i: (i, 0)),
            pl.BlockSpec((NC, BR, D), lambda i: (0, i, 0)),
            pl.BlockSpec((BR, D), lambda i: (i, 0)),
            pl.BlockSpec((D, D), lambda i: (0, 0)),
            pl.BlockSpec((1, D), lambda i: (0, 0)),
            pl.BlockSpec((D, D), lambda i: (0, 0)),
        ],
        out_specs=[
            pl.BlockSpec((BR, D), lambda i: (i, 0)),
            pl.BlockSpec((BR, D), lambda i: (i, 0)),
        ],
        out_shape=[
            jax.ShapeDtypeStruct((N, D), jnp.float32),
            jax.ShapeDtypeStruct((N, D), jnp.float32),
        ],
    )(h, partials, dinv, wst, b, wnt_next)


def _final_body(h_ref, p_ref, dinv_ref, wst_ref, b_ref, out_ref):
    p = p_ref[0] + p_ref[1]
    out_ref[...] = (_dot(h_ref[...], wst_ref[...]) + p * dinv_ref[...]
                    + b_ref[...])


def _final(h, partials, dinv, wst, b):
    return pl.pallas_call(
        _final_body,
        grid=(N // BR,),
        in_specs=[
            pl.BlockSpec((BR, D), lambda i: (i, 0)),
            pl.BlockSpec((NC, BR, D), lambda i: (0, i, 0)),
            pl.BlockSpec((BR, D), lambda i: (i, 0)),
            pl.BlockSpec((D, D), lambda i: (0, 0)),
            pl.BlockSpec((1, D), lambda i: (0, 0)),
        ],
        out_specs=pl.BlockSpec((BR, D), lambda i: (i, 0)),
        out_shape=jax.ShapeDtypeStruct((N, D), jnp.float32),
    )(h, partials, dinv, wst, b)


def kernel(x, edge_index, W_self0, W_neigh0, b0,
           W_self1, W_neigh1, b1, W_self2, W_neigh2, b2):
    src = edge_index[0]
    dst = edge_index[1]
    zeros = jnp.zeros((NP, D), jnp.float32)
    ones = jnp.ones((K, D), jnp.float32)
    src, dst, zeros, ones = lax.optimization_barrier((src, dst, zeros, ones))

    sc_agg = _agg_kernel()
    sc_agg_deg = _agg_deg_kernel()

    t0 = _table0(x, W_neigh0.T)
    p1, pd = sc_agg_deg(src, dst, t0, zeros, ones)
    h1, t1, dinv = _combine1(x, p1, pd, W_self0.T, b0.reshape(1, D),
                             W_neigh1.T)
    p2 = sc_agg(src, dst, t1, zeros)
    h2, t2 = _combine2(h1, p2, dinv, W_self1.T, b1.reshape(1, D), W_neigh2.T)
    p3 = sc_agg(src, dst, t2, zeros)
    return _final(h2, p3, dinv, W_self2.T, b2.reshape(1, D))


# speedup vs baseline: 5.6290x; 1.2300x over previous
import functools

import jax
import jax.numpy as jnp
from jax import lax
from jax.experimental import pallas as pl
from jax.experimental.pallas import tpu as pltpu
from jax.experimental.pallas import tpu_sc as plsc

N = 10000
NP = 10112
E = 320000
D = 128
K = 40
NC = 2
NS = 16
NBUF = 5
BR = 1000

_PER_TILE = E // (NC * NS)
_NCHUNK = _PER_TILE // K
_NGROUP = _NCHUNK // NBUF
_RPS = NP // NS


def _sc_mesh():
    return plsc.VectorSubcoreMesh(core_axis_name="c", subcore_axis_name="s")


def _feature_phase(src_hbm, dst_hbm, table_hbm, srcv, dstv, rows, acc,
                   gsem, ssem, base0):
    def idx_load(i, b):
        base = base0 + i * K
        pltpu.sync_copy(src_hbm.at[pl.ds(base, K)], srcv[b])
        pltpu.sync_copy(dst_hbm.at[pl.ds(base, K)], dstv[b])

    def gather_start(b):
        pltpu.async_copy(table_hbm.at[srcv[b]], rows.at[b], gsem)

    def gather_wait(b):
        pltpu.make_async_copy(table_hbm.at[srcv[b]], rows.at[b], gsem).wait()

    def scat_start(b):
        pltpu.async_copy(rows.at[b], acc.at[dstv[b]], ssem, add=True)

    def scat_wait(b):
        pltpu.make_async_copy(rows.at[b], acc.at[dstv[b]], ssem).wait()

    for b in range(NBUF):
        idx_load(b, b)
        gather_start(b)
        if b >= 1:
            gather_wait(b - 1)
            scat_start(b - 1)

    def group(g, carry):
        for b in range(NBUF):
            scat_wait(b)
            idx_load(g * NBUF + b, b)
            gather_start(b)
            pb = (b - 1) % NBUF
            gather_wait(pb)
            scat_start(pb)
        return carry

    lax.fori_loop(1, _NGROUP, group, 0)

    gather_wait(NBUF - 1)
    scat_start(NBUF - 1)
    for b in range(NBUF):
        scat_wait(b)


def _deg_phase(dst_hbm, dstv, ones_v, acc, ssem, base0):
    def scat_start(b):
        pltpu.async_copy(ones_v, acc.at[dstv[b]], ssem, add=True)

    def scat_wait(b):
        pltpu.make_async_copy(ones_v, acc.at[dstv[b]], ssem).wait()

    for b in range(NBUF):
        pltpu.sync_copy(dst_hbm.at[pl.ds(base0 + b * K, K)], dstv[b])
        scat_start(b)

    def group(g, carry):
        for b in range(NBUF):
            scat_wait(b)
            base = base0 + (g * NBUF + b) * K
            pltpu.sync_copy(dst_hbm.at[pl.ds(base, K)], dstv[b])
            scat_start(b)
        return carry

    lax.fori_loop(1, _NGROUP, group, 0)
    for b in range(NBUF):
        scat_wait(b)


_AGG_SCRATCH = (
    [pltpu.VMEM((K,), jnp.int32) for _ in range(2 * NBUF)]
    + [
        pltpu.VMEM((NBUF, K, D), jnp.float32),
        pltpu.VMEM_SHARED((NP, D), jnp.float32),
        pltpu.SemaphoreType.DMA,
        pltpu.SemaphoreType.DMA,
    ]
)


def _agg_kernel():
    @functools.partial(
        pl.kernel,
        mesh=_sc_mesh(),
        out_type=jax.ShapeDtypeStruct((NC, NP, D), jnp.float32),
        scratch_types=list(_AGG_SCRATCH),
    )
    def k(src_hbm, dst_hbm, table_hbm, zeros_hbm, out_hbm,
          s0, s1, s2, s3, s4, d0, d1, d2, d3, d4,
          rows, acc, gsem, ssem):
        srcv = (s0, s1, s2, s3, s4)
        dstv = (d0, d1, d2, d3, d4)
        c = lax.axis_index("c")
        s = lax.axis_index("s")
        r0 = s * _RPS
        sl = pl.ds(r0, _RPS)
        pltpu.sync_copy(zeros_hbm.at[sl], acc.at[sl])
        plsc.subcore_barrier()
        base0 = c * (E // NC) + s * _PER_TILE
        _feature_phase(src_hbm, dst_hbm, table_hbm, srcv, dstv, rows, acc,
                       gsem, ssem, base0)
        plsc.subcore_barrier()
        pltpu.sync_copy(acc.at[sl], out_hbm.at[c, sl])

    return k


def _agg_deg_kernel():
    @functools.partial(
        pl.kernel,
        mesh=_sc_mesh(),
        out_type=[
            jax.ShapeDtypeStruct((NC, NP, D), jnp.float32),
            jax.ShapeDtypeStruct((NC, NP, D), jnp.float32),
        ],
        scratch_types=list(_AGG_SCRATCH) + [pltpu.VMEM((K, D), jnp.float32)],
    )
    def k(src_hbm, dst_hbm, table_hbm, zeros_hbm, ones_hbm,
          out_hbm, outdeg_hbm,
          s0, s1, s2, s3, s4, d0, d1, d2, d3, d4,
          rows, acc, gsem, ssem, ones_v):
        srcv = (s0, s1, s2, s3, s4)
        dstv = (d0, d1, d2, d3, d4)
        c = lax.axis_index("c")
        s = lax.axis_index("s")
        r0 = s * _RPS
        sl = pl.ds(r0, _RPS)
        pltpu.sync_copy(zeros_hbm.at[sl], acc.at[sl])
        pltpu.sync_copy(ones_hbm, ones_v)
        plsc.subcore_barrier()
        base0 = c * (E // NC) + s * _PER_TILE
        _feature_phase(src_hbm, dst_hbm, table_hbm, srcv, dstv, rows, acc,
                       gsem, ssem, base0)
        plsc.subcore_barrier()
        pltpu.sync_copy(acc.at[sl], out_hbm.at[c, sl])
        pltpu.sync_copy(zeros_hbm.at[sl], acc.at[sl])
        plsc.subcore_barrier()
        _deg_phase(dst_hbm, dstv, ones_v, acc, ssem, base0)
        plsc.subcore_barrier()
        pltpu.sync_copy(acc.at[sl], outdeg_hbm.at[c, sl])

    return k


def _dot(a, b):
    return jnp.dot(a, b, preferred_element_type=jnp.float32,
                   precision=lax.Precision.HIGHEST)


def _table0_body(x_ref, wnt_ref, out_ref):
    out_ref[...] = _dot(x_ref[...], wnt_ref[...])


def _table0(x, wnt):
    return pl.pallas_call(
        _table0_body,
        grid=(N // BR,),
        in_specs=[
            pl.BlockSpec((BR, D), lambda i: (i, 0)),
            pl.BlockSpec((D, D), lambda i: (0, 0)),
        ],
        out_specs=pl.BlockSpec((BR, D), lambda i: (i, 0)),
        out_shape=jax.ShapeDtypeStruct((N, D), jnp.float32),
    )(x, wnt)


def _combine1_body(h_ref, p_ref, pd_ref, wst_ref, b_ref, wnt_ref,
                   h_out_ref, t_out_ref, dinv_out_ref):
    deginv = 1.0 / jnp.maximum(pd_ref[0] + pd_ref[1], 1.0)
    dinv_out_ref[...] = deginv
    p = p_ref[0] + p_ref[1]
    z = _dot(h_ref[...], wst_ref[...]) + p * deginv + b_ref[...]
    h_next = jnp.maximum(z, 0.0)
    h_out_ref[...] = h_next
    t_out_ref[...] = _dot(h_next, wnt_ref[...])


def _combine1(h, partials, deg_partials, wst, b, wnt_next):
    return pl.pallas_call(
        _combine1_body,
        grid=(N // BR,),
        in_specs=[
            pl.BlockSpec((BR, D), lambda i: (i, 0)),
            pl.BlockSpec((NC, BR, D), lambda i: (0, i, 0)),
            pl.BlockSpec((NC, BR, D), lambda i: (0, i, 0)),
            pl.BlockSpec((D, D), lambda i: (0, 0)),
            pl.BlockSpec((1, D), lambda i: (0, 0)),
            pl.BlockSpec((D, D), lambda i: (0, 0)),
        ],
        out_specs=[
            pl.BlockSpec((BR, D), lambda i: (i, 0)),
            pl.BlockSpec((BR, D), lambda i: (i, 0)),
            pl.BlockSpec((BR, D), lambda i: (i, 0)),
        ],
        out_shape=[
            jax.ShapeDtypeStruct((N, D), jnp.float32),
            jax.ShapeDtypeStruct((N, D), jnp.float32),
            jax.ShapeDtypeStruct((N, D), jnp.float32),
        ],
    )(h, partials, deg_partials, wst, b, wnt_next)


def _combine2_body(h_ref, p_ref, dinv_ref, wst_ref, b_ref, wnt_ref,
                   h_out_ref, t_out_ref):
    p = p_ref[0] + p_ref[1]
    z = _dot(h_ref[...], wst_ref[...]) + p * dinv_ref[...] + b_ref[...]
    h_next = jnp.maximum(z, 0.0)
    h_out_ref[...] = h_next
    t_out_ref[...] = _dot(h_next, wnt_ref[...])


def _combine2(h, partials, dinv, wst, b, wnt_next):
    return pl.pallas_call(
        _combine2_body,
        grid=(N // BR,),
        in_specs=[
            pl.BlockSpec((BR, D), lambda i: (i, 0)),
            pl.BlockSpec((NC, BR, D), lambda i: (0, i, 0)),
            pl.BlockSpec((BR, D), lambda i: (i, 0)),
            pl.BlockSpec((D, D), lambda i: (0, 0)),
            pl.BlockSpec((1, D), lambda i: (0, 0)),
            pl.BlockSpec((D, D), lambda i: (0, 0)),
        ],
        out_specs=[
            pl.BlockSpec((BR, D), lambda i: (i, 0)),
            pl.BlockSpec((BR, D), lambda i: (i, 0)),
        ],
        out_shape=[
            jax.ShapeDtypeStruct((N, D), jnp.float32),
            jax.ShapeDtypeStruct((N, D), jnp.float32),
        ],
    )(h, partials, dinv, wst, b, wnt_next)


def _final_body(h_ref, p_ref, dinv_ref, wst_ref, b_ref, out_ref):
    p = p_ref[0] + p_ref[1]
    out_ref[...] = (_dot(h_ref[...], wst_ref[...]) + p * dinv_ref[...]
                    + b_ref[...])


def _final(h, partials, dinv, wst, b):
    return pl.pallas_call(
        _final_body,
        grid=(N // BR,),
        in_specs=[
            pl.BlockSpec((BR, D), lambda i: (i, 0)),
            pl.BlockSpec((NC, BR, D), lambda i: (0, i, 0)),
            pl.BlockSpec((BR, D), lambda i: (i, 0)),
            pl.BlockSpec((D, D), lambda i: (0, 0)),
            pl.BlockSpec((1, D), lambda i: (0, 0)),
        ],
        out_specs=pl.BlockSpec((BR, D), lambda i: (i, 0)),
        out_shape=jax.ShapeDtypeStruct((N, D), jnp.float32),
    )(h, partials, dinv, wst, b)


def kernel(x, edge_index, W_self0, W_neigh0, b0,
           W_self1, W_neigh1, b1, W_self2, W_neigh2, b2):
    src = edge_index[0]
    dst = edge_index[1]
    zeros = jnp.zeros((NP, D), jnp.float32)
    ones = jnp.ones((K, D), jnp.float32)
    src, dst, zeros, ones = lax.optimization_barrier((src, dst, zeros, ones))

    sc_agg = _agg_kernel()
    sc_agg_deg = _agg_deg_kernel()

    t0 = _table0(x, W_neigh0.T)
    p1, pd = sc_agg_deg(src, dst, t0, zeros, ones)
    h1, t1, dinv = _combine1(x, p1, pd, W_self0.T, b0.reshape(1, D),
                             W_neigh1.T)
    p2 = sc_agg(src, dst, t1, zeros)
    h2, t2 = _combine2(h1, p2, dinv, W_self1.T, b1.reshape(1, D), W_neigh2.T)
    p3 = sc_agg(src, dst, t2, zeros)
    return _final(h2, p3, dinv, W_self2.T, b2.reshape(1, D))


# speedup vs baseline: 11.4251x; 2.0297x over previous
import functools

import jax
import jax.numpy as jnp
from jax import lax
from jax.experimental import pallas as pl
from jax.experimental.pallas import tpu as pltpu
from jax.experimental.pallas import tpu_sc as plsc

N = 10000
NP = 10112
E = 320000
D = 128
K = 128
KT = 16
K0 = 80
NC = 2
NS = 16
BR = 1000
DH = 80

_PER_TILE = E // (NC * NS)
_RPS = NP // NS


def _sc_mesh():
    return plsc.VectorSubcoreMesh(core_axis_name="c", subcore_axis_name="s")


def _hist_update(deghist, dslot, k):
    ones16 = jnp.ones((16,), jnp.float32)
    for j in range(k // 16):
        d16 = dslot[pl.ds(j * 16, 16)]
        plsc.addupdate_scatter(
            deghist,
            [jax.lax.shift_right_logical(d16, 7),
             jax.lax.bitwise_and(d16, 127)],
            ones16)


def _feature_phase(dst_hbm, table_hbm, srcall, dstv, rows, acc,
                   isem, gsem, ssem, base0, k, nfull, deghist=None,
                   dstt=None, rowt=None):
    def dst_start(i, b):
        pltpu.async_copy(dst_hbm.at[pl.ds(base0 + i * k, k)], dstv[b], isem)

    def dst_wait(i, b):
        pltpu.make_async_copy(dst_hbm.at[pl.ds(base0 + i * k, k)],
                              dstv[b], isem).wait()

    def gather_start(i, b):
        pltpu.async_copy(table_hbm.at[srcall.at[pl.ds(i * k, k)]],
                         rows.at[b], gsem)

    def gather_wait(i, b):
        pltpu.make_async_copy(table_hbm.at[srcall.at[pl.ds(i * k, k)]],
                              rows.at[b], gsem).wait()

    def scat_start(b):
        pltpu.async_copy(rows.at[b], acc.at[dstv[b]], ssem, add=True)

    def scat_wait(b):
        pltpu.make_async_copy(rows.at[b], acc.at[dstv[b]], ssem).wait()

    def hist(b):
        if deghist is not None:
            _hist_update(deghist, dstv[b], k)

    def step(i, b):
        scat_wait(b)
        dst_start(i, b)
        gather_start(i, b)
        gather_wait(i - 1, 1 - b)
        dst_wait(i - 1, 1 - b)
        hist(1 - b)
        scat_start(1 - b)

    dst_start(0, 0)
    gather_start(0, 0)
    dst_start(1, 1)
    gather_start(1, 1)
    gather_wait(0, 0)
    dst_wait(0, 0)
    hist(0)
    scat_start(0)

    def group(g, carry):
        step(2 * g + 2, 0)
        step(2 * g + 3, 1)
        return carry

    lax.fori_loop(0, (nfull - 2) // 2, group, 0)

    last = nfull - 1
    if nfull % 2 == 1:
        step(last, 0)
        lb = 0
    else:
        lb = 1
    gather_wait(last, lb)
    dst_wait(last, lb)
    hist(lb)
    scat_start(lb)
    scat_wait(1 - lb)
    scat_wait(lb)

    if dstt is not None:
        tbase = base0 + nfull * k
        pltpu.sync_copy(dst_hbm.at[pl.ds(tbase, KT)], dstt)
        pltpu.async_copy(table_hbm.at[srcall.at[pl.ds(nfull * k, KT)]],
                         rowt, gsem).wait()
        if deghist is not None:
            _hist_update(deghist, dstt, KT)
        pltpu.sync_copy(rowt, acc.at[dstt], add=True)


def _agg_kernel():
    @functools.partial(
        pl.kernel,
        mesh=_sc_mesh(),
        out_type=jax.ShapeDtypeStruct((NC, NP, D), jnp.float32),
        scratch_types=[
            pltpu.VMEM((_PER_TILE,), jnp.int32),
            pltpu.VMEM((K,), jnp.int32),
            pltpu.VMEM((K,), jnp.int32),
            pltpu.VMEM((KT,), jnp.int32),
            pltpu.VMEM((2, K, D), jnp.float32),
            pltpu.VMEM((KT, D), jnp.float32),
            pltpu.VMEM_SHARED((NP, D), jnp.float32),
            pltpu.SemaphoreType.DMA,
            pltpu.SemaphoreType.DMA,
            pltpu.SemaphoreType.DMA,
        ],
    )
    def k(src_hbm, dst_hbm, table_hbm, zeros_hbm, out_hbm,
          srcall, d0, d1, dstt, rows, rowt, acc, isem, gsem, ssem):
        c = lax.axis_index("c")
        s = lax.axis_index("s")
        r0 = s * _RPS
        sl = pl.ds(r0, _RPS)
        pltpu.sync_copy(zeros_hbm.at[sl], acc.at[sl])
        base0 = c * (E // NC) + s * _PER_TILE
        pltpu.sync_copy(src_hbm.at[pl.ds(base0, _PER_TILE)], srcall)
        plsc.subcore_barrier()
        _feature_phase(dst_hbm, table_hbm, srcall, (d0, d1), rows, acc,
                       isem, gsem, ssem, base0, K, _PER_TILE // K,
                       dstt=dstt, rowt=rowt)
        plsc.subcore_barrier()
        pltpu.sync_copy(acc.at[sl], out_hbm.at[c, sl])

    return k


def _agg_deg_kernel():
    @functools.partial(
        pl.kernel,
        mesh=_sc_mesh(),
        compiler_params=pltpu.CompilerParams(needs_layout_passes=False),
        out_type=[
            jax.ShapeDtypeStruct((NC, NP, D), jnp.float32),
            jax.ShapeDtypeStruct((NC, DH, D), jnp.float32),
        ],
        scratch_types=[
            pltpu.VMEM((_PER_TILE,), jnp.int32),
            pltpu.VMEM((K0,), jnp.int32),
            pltpu.VMEM((K0,), jnp.int32),
            pltpu.VMEM((2, K0, D), jnp.float32),
            pltpu.VMEM((DH, D), jnp.float32),
            pltpu.VMEM((DH,), jnp.int32),
            pltpu.VMEM_SHARED((NP, D), jnp.float32),
            pltpu.VMEM_SHARED((DH, D), jnp.float32),
            pltpu.SemaphoreType.DMA,
            pltpu.SemaphoreType.DMA,
            pltpu.SemaphoreType.DMA,
        ],
    )
    def k(src_hbm, dst_hbm, table_hbm, zeros_hbm, iota_hbm,
          out_hbm, outdeg_hbm,
          srcall, d0, d1, rows, deghist, iotav, acc, degacc,
          isem, gsem, ssem):
        c = lax.axis_index("c")
        s = lax.axis_index("s")
        r0 = s * _RPS
        sl = pl.ds(r0, _RPS)
        pltpu.sync_copy(zeros_hbm.at[sl], acc.at[sl])
        pltpu.sync_copy(zeros_hbm.at[pl.ds(0, DH)], deghist)
        pltpu.sync_copy(iota_hbm, iotav)
        base0 = c * (E // NC) + s * _PER_TILE
        pltpu.sync_copy(src_hbm.at[pl.ds(base0, _PER_TILE)], srcall)

        @pl.when(s < DH // 8)
        def _():
            dsl = pl.ds(s * 8, 8)
            pltpu.sync_copy(zeros_hbm.at[dsl], degacc.at[dsl])

        plsc.subcore_barrier()
        _feature_phase(dst_hbm, table_hbm, srcall, (d0, d1), rows, acc,
                       isem, gsem, ssem, base0, K0, _PER_TILE // K0,
                       deghist=deghist)
        plsc.subcore_barrier()
        pltpu.sync_copy(acc.at[sl], out_hbm.at[c, sl])
        pltpu.sync_copy(deghist, degacc.at[iotav], add=True)
        plsc.subcore_barrier()

        @pl.when(s < DH // 8)
        def _():
            dsl = pl.ds(s * 8, 8)
            pltpu.sync_copy(degacc.at[dsl], outdeg_hbm.at[c, dsl])

    return k


def _dot(a, b):
    return jnp.dot(a, b, preferred_element_type=jnp.float32,
                   precision=lax.Precision.HIGHEST)


def _table0_body(x_ref, wnt_ref, out_ref):
    out_ref[...] = _dot(x_ref[...], wnt_ref[...])


def _table0(x, wnt):
    return pl.pallas_call(
        _table0_body,
        grid=(N // BR,),
        in_specs=[
            pl.BlockSpec((BR, D), lambda i: (i, 0)),
            pl.BlockSpec((D, D), lambda i: (0, 0)),
        ],
        out_specs=pl.BlockSpec((BR, D), lambda i: (i, 0)),
        out_shape=jax.ShapeDtypeStruct((N, D), jnp.float32),
    )(x, wnt)


def _deginv_body(dp_ref, out_ref):
    d = dp_ref[0] + dp_ref[1]
    out_ref[...] = 1.0 / jnp.maximum(d, 1.0)


def _deginv(deg_partials):
    out = pl.pallas_call(
        _deginv_body,
        grid=(1,),
        in_specs=[pl.BlockSpec((NC, DH, D), lambda i: (0, 0, 0))],
        out_specs=pl.BlockSpec((DH, D), lambda i: (0, 0)),
        out_shape=jax.ShapeDtypeStruct((DH, D), jnp.float32),
    )(deg_partials)
    return out.reshape(DH * D, 1)


def _combine_body(h_ref, p_ref, dinv_ref, wst_ref, b_ref, wnt_ref,
                  h_out_ref, t_out_ref):
    p = p_ref[0] + p_ref[1]
    z = _dot(h_ref[...], wst_ref[...]) + p * dinv_ref[...] + b_ref[...]
    h_next = jnp.maximum(z, 0.0)
    h_out_ref[...] = h_next
    t_out_ref[...] = _dot(h_next, wnt_ref[...])


def _combine(h, partials, dinv, wst, b, wnt_next):
    return pl.pallas_call(
        _combine_body,
        grid=(N // BR,),
        in_specs=[
            pl.BlockSpec((BR, D), lambda i: (i, 0)),
            pl.BlockSpec((NC, BR, D), lambda i: (0, i, 0)),
            pl.BlockSpec((BR, 1), lambda i: (i, 0)),
            pl.BlockSpec((D, D), lambda i: (0, 0)),
            pl.BlockSpec((1, D), lambda i: (0, 0)),
            pl.BlockSpec((D, D), lambda i: (0, 0)),
        ],
        out_specs=[
            pl.BlockSpec((BR, D), lambda i: (i, 0)),
            pl.BlockSpec((BR, D), lambda i: (i, 0)),
        ],
        out_shape=[
            jax.ShapeDtypeStruct((N, D), jnp.float32),
            jax.ShapeDtypeStruct((N, D), jnp.float32),
        ],
    )(h, partials, dinv, wst, b, wnt_next)


def _final_body(h_ref, p_ref, dinv_ref, wst_ref, b_ref, out_ref):
    p = p_ref[0] + p_ref[1]
    out_ref[...] = (_dot(h_ref[...], wst_ref[...]) + p * dinv_ref[...]
                    + b_ref[...])


def _final(h, partials, dinv, wst, b):
    return pl.pallas_call(
        _final_body,
        grid=(N // BR,),
        in_specs=[
            pl.BlockSpec((BR, D), lambda i: (i, 0)),
            pl.BlockSpec((NC, BR, D), lambda i: (0, i, 0)),
            pl.BlockSpec((BR, 1), lambda i: (i, 0)),
            pl.BlockSpec((D, D), lambda i: (0, 0)),
            pl.BlockSpec((1, D), lambda i: (0, 0)),
        ],
        out_specs=pl.BlockSpec((BR, D), lambda i: (i, 0)),
        out_shape=jax.ShapeDtypeStruct((N, D), jnp.float32),
    )(h, partials, dinv, wst, b)


def kernel(x, edge_index, W_self0, W_neigh0, b0,
           W_self1, W_neigh1, b1, W_self2, W_neigh2, b2):
    src = edge_index[0]
    dst = edge_index[1]
    zeros = jnp.zeros((NP, D), jnp.float32)
    iota = jnp.arange(DH, dtype=jnp.int32)
    src, dst, zeros, iota = lax.optimization_barrier((src, dst, zeros, iota))

    sc_agg = _agg_kernel()
    sc_agg_deg = _agg_deg_kernel()

    t0 = _table0(x, W_neigh0.T)
    p1, dp = sc_agg_deg(src, dst, t0, zeros, iota)
    dinv = _deginv(dp)
    h1, t1 = _combine(x, p1, dinv, W_self0.T, b0.reshape(1, D), W_neigh1.T)
    p2 = sc_agg(src, dst, t1, zeros)
    h2, t2 = _combine(h1, p2, dinv, W_self1.T, b1.reshape(1, D), W_neigh2.T)
    p3 = sc_agg(src, dst, t2, zeros)
    return _final(h2, p3, dinv, W_self2.T, b2.reshape(1, D))


# speedup vs baseline: 12.9578x; 1.1342x over previous
import functools

import jax
import jax.numpy as jnp
from jax import lax
from jax.experimental import pallas as pl
from jax.experimental.pallas import tpu as pltpu
from jax.experimental.pallas import tpu_sc as plsc

N = 10000
NP = 10112
E = 320000
D = 128
K = 128
KT = 16
K0 = 80
NC = 2
NS = 16
BR = 2000
DH = 80

_PER_TILE = E // (NC * NS)
_RPS = NP // NS


def _sc_mesh():
    return plsc.VectorSubcoreMesh(core_axis_name="c", subcore_axis_name="s")


def _hist_update(deghist, dslot, k):
    ones16 = jnp.ones((16,), jnp.float32)
    for j in range(k // 16):
        d16 = dslot[pl.ds(j * 16, 16)]
        plsc.addupdate_scatter(
            deghist,
            [jax.lax.shift_right_logical(d16, 7),
             jax.lax.bitwise_and(d16, 127)],
            ones16)


def _feature_phase(dst_hbm, table_hbm, srcall, dstv, rows, acc,
                   isem, gsem, ssem, base0, k, nfull, deghist=None,
                   dstt=None, rowt=None):
    def dst_start(i, b):
        pltpu.async_copy(dst_hbm.at[pl.ds(base0 + i * k, k)], dstv[b], isem)

    def dst_wait(i, b):
        pltpu.make_async_copy(dst_hbm.at[pl.ds(base0 + i * k, k)],
                              dstv[b], isem).wait()

    def gather_start(i, b):
        pltpu.async_copy(table_hbm.at[srcall.at[pl.ds(i * k, k)]],
                         rows.at[b], gsem)

    def gather_wait(i, b):
        pltpu.make_async_copy(table_hbm.at[srcall.at[pl.ds(i * k, k)]],
                              rows.at[b], gsem).wait()

    def scat_start(b):
        pltpu.async_copy(rows.at[b], acc.at[dstv[b]], ssem, add=True)

    def scat_wait(b):
        pltpu.make_async_copy(rows.at[b], acc.at[dstv[b]], ssem).wait()

    def hist(b):
        if deghist is not None:
            _hist_update(deghist, dstv[b], k)

    def step(i, b):
        scat_wait(b)
        dst_start(i, b)
        gather_start(i, b)
        gather_wait(i - 1, 1 - b)
        dst_wait(i - 1, 1 - b)
        hist(1 - b)
        scat_start(1 - b)

    dst_start(0, 0)
    gather_start(0, 0)
    dst_start(1, 1)
    gather_start(1, 1)
    gather_wait(0, 0)
    dst_wait(0, 0)
    hist(0)
    scat_start(0)

    def group(g, carry):
        step(2 * g + 2, 0)
        step(2 * g + 3, 1)
        return carry

    lax.fori_loop(0, (nfull - 2) // 2, group, 0)

    last = nfull - 1
    if nfull % 2 == 1:
        step(last, 0)
        lb = 0
    else:
        lb = 1
    gather_wait(last, lb)
    dst_wait(last, lb)
    hist(lb)
    scat_start(lb)
    scat_wait(1 - lb)
    scat_wait(lb)

    if dstt is not None:
        tbase = base0 + nfull * k
        pltpu.sync_copy(dst_hbm.at[pl.ds(tbase, KT)], dstt)
        pltpu.async_copy(table_hbm.at[srcall.at[pl.ds(nfull * k, KT)]],
                         rowt, gsem).wait()
        if deghist is not None:
            _hist_update(deghist, dstt, KT)
        pltpu.sync_copy(rowt, acc.at[dstt], add=True)


def _agg_kernel():
    @functools.partial(
        pl.kernel,
        mesh=_sc_mesh(),
        out_type=jax.ShapeDtypeStruct((NC, NP, D), jnp.float32),
        scratch_types=[
            pltpu.VMEM((_PER_TILE,), jnp.int32),
            pltpu.VMEM((K,), jnp.int32),
            pltpu.VMEM((K,), jnp.int32),
            pltpu.VMEM((KT,), jnp.int32),
            pltpu.VMEM((2, K, D), jnp.float32),
            pltpu.VMEM((KT, D), jnp.float32),
            pltpu.VMEM_SHARED((NP, D), jnp.float32),
            pltpu.SemaphoreType.DMA,
            pltpu.SemaphoreType.DMA,
            pltpu.SemaphoreType.DMA,
        ],
    )
    def k(src_hbm, dst_hbm, table_hbm, zeros_hbm, out_hbm,
          srcall, d0, d1, dstt, rows, rowt, acc, isem, gsem, ssem):
        c = lax.axis_index("c")
        s = lax.axis_index("s")
        r0 = s * _RPS
        sl = pl.ds(r0, _RPS)
        pltpu.sync_copy(zeros_hbm.at[sl], acc.at[sl])
        base0 = c * (E // NC) + s * _PER_TILE
        pltpu.sync_copy(src_hbm.at[pl.ds(base0, _PER_TILE)], srcall)
        plsc.subcore_barrier()
        _feature_phase(dst_hbm, table_hbm, srcall, (d0, d1), rows, acc,
                       isem, gsem, ssem, base0, K, _PER_TILE // K,
                       dstt=dstt, rowt=rowt)
        plsc.subcore_barrier()
        pltpu.sync_copy(acc.at[sl], out_hbm.at[c, sl])

    return k


def _agg_deg_kernel():
    @functools.partial(
        pl.kernel,
        mesh=_sc_mesh(),
        compiler_params=pltpu.CompilerParams(needs_layout_passes=False),
        out_type=[
            jax.ShapeDtypeStruct((NC, NP, D), jnp.float32),
            jax.ShapeDtypeStruct((NC, DH, D), jnp.float32),
        ],
        scratch_types=[
            pltpu.VMEM((_PER_TILE,), jnp.int32),
            pltpu.VMEM((K0,), jnp.int32),
            pltpu.VMEM((K0,), jnp.int32),
            pltpu.VMEM((2, K0, D), jnp.float32),
            pltpu.VMEM((DH, D), jnp.float32),
            pltpu.VMEM((DH,), jnp.int32),
            pltpu.VMEM_SHARED((NP, D), jnp.float32),
            pltpu.VMEM_SHARED((DH, D), jnp.float32),
            pltpu.SemaphoreType.DMA,
            pltpu.SemaphoreType.DMA,
            pltpu.SemaphoreType.DMA,
        ],
    )
    def k(src_hbm, dst_hbm, table_hbm, zeros_hbm, iota_hbm,
          out_hbm, outdeg_hbm,
          srcall, d0, d1, rows, deghist, iotav, acc, degacc,
          isem, gsem, ssem):
        c = lax.axis_index("c")
        s = lax.axis_index("s")
        r0 = s * _RPS
        sl = pl.ds(r0, _RPS)
        pltpu.sync_copy(zeros_hbm.at[sl], acc.at[sl])
        pltpu.sync_copy(zeros_hbm.at[pl.ds(0, DH)], deghist)
        pltpu.sync_copy(iota_hbm, iotav)
        base0 = c * (E // NC) + s * _PER_TILE
        pltpu.sync_copy(src_hbm.at[pl.ds(base0, _PER_TILE)], srcall)

        @pl.when(s < DH // 8)
        def _():
            dsl = pl.ds(s * 8, 8)
            pltpu.sync_copy(zeros_hbm.at[dsl], degacc.at[dsl])

        plsc.subcore_barrier()
        _feature_phase(dst_hbm, table_hbm, srcall, (d0, d1), rows, acc,
                       isem, gsem, ssem, base0, K0, _PER_TILE // K0,
                       deghist=deghist)
        plsc.subcore_barrier()
        pltpu.sync_copy(acc.at[sl], out_hbm.at[c, sl])
        pltpu.sync_copy(deghist, degacc.at[iotav], add=True)
        plsc.subcore_barrier()

        @pl.when(s < DH // 8)
        def _():
            dsl = pl.ds(s * 8, 8)
            pltpu.sync_copy(degacc.at[dsl], outdeg_hbm.at[c, dsl])

    return k


def _dot(a, b):
    return jnp.dot(a, b, preferred_element_type=jnp.float32)


def _table0_body(x_ref, wnt_ref, out_ref):
    out_ref[...] = _dot(x_ref[...], wnt_ref[...])


def _table0(x, wnt):
    return pl.pallas_call(
        _table0_body,
        grid=(N // BR,),
        in_specs=[
            pl.BlockSpec((BR, D), lambda i: (i, 0)),
            pl.BlockSpec((D, D), lambda i: (0, 0)),
        ],
        out_specs=pl.BlockSpec((BR, D), lambda i: (i, 0)),
        out_shape=jax.ShapeDtypeStruct((N, D), jnp.float32),
    )(x, wnt)


def _deginv_body(dp_ref, out_ref):
    d = dp_ref[0] + dp_ref[1]
    out_ref[...] = 1.0 / jnp.maximum(d, 1.0)


def _deginv(deg_partials):
    out = pl.pallas_call(
        _deginv_body,
        grid=(1,),
        in_specs=[pl.BlockSpec((NC, DH, D), lambda i: (0, 0, 0))],
        out_specs=pl.BlockSpec((DH, D), lambda i: (0, 0)),
        out_shape=jax.ShapeDtypeStruct((DH, D), jnp.float32),
    )(deg_partials)
    return out.reshape(DH * D, 1)


def _combine_body(h_ref, p_ref, dinv_ref, wst_ref, b_ref, wnt_ref,
                  h_out_ref, t_out_ref):
    p = p_ref[0] + p_ref[1]
    z = _dot(h_ref[...], wst_ref[...]) + p * dinv_ref[...] + b_ref[...]
    h_next = jnp.maximum(z, 0.0)
    h_out_ref[...] = h_next
    t_out_ref[...] = _dot(h_next, wnt_ref[...])


def _combine(h, partials, dinv, wst, b, wnt_next):
    return pl.pallas_call(
        _combine_body,
        grid=(N // BR,),
        in_specs=[
            pl.BlockSpec((BR, D), lambda i: (i, 0)),
            pl.BlockSpec((NC, BR, D), lambda i: (0, i, 0)),
            pl.BlockSpec((BR, 1), lambda i: (i, 0)),
            pl.BlockSpec((D, D), lambda i: (0, 0)),
            pl.BlockSpec((1, D), lambda i: (0, 0)),
            pl.BlockSpec((D, D), lambda i: (0, 0)),
        ],
        out_specs=[
            pl.BlockSpec((BR, D), lambda i: (i, 0)),
            pl.BlockSpec((BR, D), lambda i: (i, 0)),
        ],
        out_shape=[
            jax.ShapeDtypeStruct((N, D), jnp.float32),
            jax.ShapeDtypeStruct((N, D), jnp.float32),
        ],
    )(h, partials, dinv, wst, b, wnt_next)


def _final_body(h_ref, p_ref, dinv_ref, wst_ref, b_ref, out_ref):
    p = p_ref[0] + p_ref[1]
    out_ref[...] = (_dot(h_ref[...], wst_ref[...]) + p * dinv_ref[...]
                    + b_ref[...])


def _final(h, partials, dinv, wst, b):
    return pl.pallas_call(
        _final_body,
        grid=(N // BR,),
        in_specs=[
            pl.BlockSpec((BR, D), lambda i: (i, 0)),
            pl.BlockSpec((NC, BR, D), lambda i: (0, i, 0)),
            pl.BlockSpec((BR, 1), lambda i: (i, 0)),
            pl.BlockSpec((D, D), lambda i: (0, 0)),
            pl.BlockSpec((1, D), lambda i: (0, 0)),
        ],
        out_specs=pl.BlockSpec((BR, D), lambda i: (i, 0)),
        out_shape=jax.ShapeDtypeStruct((N, D), jnp.float32),
    )(h, partials, dinv, wst, b)


def kernel(x, edge_index, W_self0, W_neigh0, b0,
           W_self1, W_neigh1, b1, W_self2, W_neigh2, b2):
    src = edge_index[0]
    dst = edge_index[1]
    zeros = jnp.zeros((NP, D), jnp.float32)
    iota = jnp.arange(DH, dtype=jnp.int32)
    src, dst, zeros, iota = lax.optimization_barrier((src, dst, zeros, iota))

    sc_agg = _agg_kernel()
    sc_agg_deg = _agg_deg_kernel()

    t0 = _table0(x, W_neigh0.T)
    p1, dp = sc_agg_deg(src, dst, t0, zeros, iota)
    dinv = _deginv(dp)
    h1, t1 = _combine(x, p1, dinv, W_self0.T, b0.reshape(1, D), W_neigh1.T)
    p2 = sc_agg(src, dst, t1, zeros)
    h2, t2 = _combine(h1, p2, dinv, W_self1.T, b1.reshape(1, D), W_neigh2.T)
    p3 = sc_agg(src, dst, t2, zeros)
    return _final(h2, p3, dinv, W_self2.T, b2.reshape(1, D))
